# speedup vs baseline: 3.2399x; 3.2399x over previous
import functools

import jax
import jax.numpy as jnp
from jax import lax
from jax.experimental import pallas as pl
from jax.experimental.pallas import tpu as pltpu
from jax.experimental.pallas import tpu_sc as plsc

D = 128
CHUNK = 128
NBUF = 2


@functools.cache
def _build(B):
    info = plsc.get_sparse_core_info()
    NC, NS = info.num_cores, info.num_subcores
    NW = NC * NS
    b_per_w = B // NW
    n_chunks = b_per_w // CHUNK
    n_outer = n_chunks // NBUF
    assert b_per_w * NW == B and n_outer * NBUF == n_chunks

    mesh = plsc.VectorSubcoreMesh(core_axis_name="c", subcore_axis_name="s")

    @functools.partial(
        pl.kernel,
        mesh=mesh,
        out_type=jax.ShapeDtypeStruct((B, D), jnp.float32),
        scratch_types=[
            pltpu.VMEM((n_chunks, CHUNK), jnp.int32),
            pltpu.VMEM((NBUF, CHUNK, D), jnp.float32),
            [pltpu.SemaphoreType.DMA] * NBUF,
            [pltpu.SemaphoreType.DMA] * NBUF,
        ],
    )
    def gather_kernel(idx_hbm, table_hbm, out_hbm, idx_v, bufs, gsems, osems):
        wid = lax.axis_index("s") * NC + lax.axis_index("c")
        base = wid * b_per_w
        pltpu.sync_copy(idx_hbm.at[wid], idx_v)

        def outer(t, _):
            for b in range(NBUF):
                j = t * NBUF + b

                @pl.when(t > 0)
                def _():
                    pltpu.make_async_copy(
                        bufs.at[b], out_hbm.at[pl.ds(0, CHUNK)], osems[b]
                    ).wait()

                pltpu.async_copy(
                    table_hbm.at[idx_v.at[j]], bufs.at[b], gsems[b]
                )
            for b in range(NBUF):
                j = t * NBUF + b
                pltpu.make_async_copy(
                    table_hbm.at[idx_v.at[j]], bufs.at[b], gsems[b]
                ).wait()
                pltpu.async_copy(
                    bufs.at[b],
                    out_hbm.at[pl.ds(base + j * CHUNK, CHUNK)],
                    osems[b],
                )
            return 0

        lax.fori_loop(0, n_outer, outer, 0, unroll=False)
        for b in range(NBUF):
            pltpu.make_async_copy(
                bufs.at[b], out_hbm.at[pl.ds(0, CHUNK)], osems[b]
            ).wait()

    return gather_kernel


def kernel(position_idx, table):
    BATCH, HIST = position_idx.shape
    B = BATCH * HIST
    info = plsc.get_sparse_core_info()
    NW = info.num_cores * info.num_subcores
    idx3 = position_idx.astype(jnp.int32).reshape(NW, (B // NW) // CHUNK, CHUNK)
    out = _build(B)(idx3, table)
    return out.reshape(BATCH, HIST, D)
